# baseline (speedup 1.0000x reference)
import functools

import jax
import jax.numpy as jnp
from jax.experimental import pallas as pl
from jax.experimental.pallas import tpu as pltpu


def _matmul_kernel(x_ref, w_ref, b_ref, o_ref):
    xb = x_ref[...].astype(jnp.bfloat16)
    o_ref[...] = (
        jnp.dot(xb, w_ref[...], preferred_element_type=jnp.float32)
        + b_ref[...]
    )


@functools.partial(jax.jit, static_argnames=())
def kernel(x, kernel, bias):
    b, v = x.shape
    n = kernel.shape[1]
    bm = 32
    w16 = kernel.astype(jnp.bfloat16)
    bias2 = bias.reshape(1, n)
    out = pl.pallas_call(
        _matmul_kernel,
        grid=(b // bm,),
        in_specs=[
            pl.BlockSpec((bm, v), lambda i: (i, 0)),
            pl.BlockSpec((v, n), lambda i: (0, 0)),
            pl.BlockSpec((1, n), lambda i: (0, 0)),
        ],
        out_specs=pl.BlockSpec((bm, n), lambda i: (i, 0)),
        out_shape=jax.ShapeDtypeStruct((b, n), jnp.float32),
        compiler_params=pltpu.CompilerParams(
            dimension_semantics=("parallel",),
        ),
    )(x, w16, bias2)
    return out
